# baseline (speedup 1.0000x reference)
import functools

import jax
import jax.numpy as jnp
from jax import lax
from jax.experimental import pallas as pl
from jax.experimental.pallas import tpu as pltpu
from jax.experimental.pallas import tpu_sc as plsc

NC = 2
NS = 16
NW = NC * NS
CHUNK = 128
ECHUNK = 64


def _cdiv(a, b):
    return (a + b - 1) // b


def _sc_mesh():
    return plsc.VectorSubcoreMesh(core_axis_name="c", subcore_axis_name="s")


def _degrees(src3, dst3, *, npad, chunks):
    rows_pt = npad // NS

    @functools.partial(
        pl.kernel,
        mesh=_sc_mesh(),
        out_type=jax.ShapeDtypeStruct((NC, 2, npad), jnp.float32),
        scratch_types=[
            pltpu.VMEM((chunks, CHUNK), jnp.int32),
            pltpu.VMEM((chunks, CHUNK), jnp.int32),
            pltpu.VMEM((CHUNK,), jnp.float32),
            pltpu.VMEM((rows_pt,), jnp.float32),
            pltpu.VMEM_SHARED((npad,), jnp.float32),
            pltpu.VMEM_SHARED((npad,), jnp.float32),
        ],
    )
    def k(src_h, dst_h, out_h, src_v, dst_v, ones_v, z_v, degs_sh, degd_sh):
        cid = lax.axis_index("c")
        sid = lax.axis_index("s")
        wid = sid * NC + cid

        @pl.loop(0, CHUNK, step=16)
        def _(i):
            ones_v[pl.ds(i, 16)] = jnp.full((16,), 1.0, jnp.float32)

        @pl.loop(0, rows_pt, step=16)
        def _(i):
            z_v[pl.ds(i, 16)] = jnp.zeros((16,), jnp.float32)

        pltpu.sync_copy(z_v, degs_sh.at[pl.ds(sid * rows_pt, rows_pt)])
        pltpu.sync_copy(z_v, degd_sh.at[pl.ds(sid * rows_pt, rows_pt)])
        plsc.subcore_barrier()

        pltpu.sync_copy(src_h.at[wid], src_v)
        pltpu.sync_copy(dst_h.at[wid], dst_v)

        @pl.loop(0, chunks)
        def _(j):
            pltpu.sync_copy(ones_v, degs_sh.at[src_v.at[j]], add=True)
            pltpu.sync_copy(ones_v, degd_sh.at[dst_v.at[j]], add=True)

        plsc.subcore_barrier()
        sl = pl.ds(sid * rows_pt, rows_pt)
        pltpu.sync_copy(degs_sh.at[sl], out_h.at[cid, 0, sl])
        pltpu.sync_copy(degd_sh.at[sl], out_h.at[cid, 1, sl])

    return k(src3, dst3)


def _edge_aggregate(src3, dst3, hws, *, npad, d, chunks):
    rows_pt = npad // NS

    nbuf = 5
    npass = 4
    assert chunks % (nbuf * npass) == 0
    half = chunks // npass

    @functools.partial(
        pl.kernel,
        mesh=_sc_mesh(),
        out_type=jax.ShapeDtypeStruct((NC, npad, d), jnp.float32),
        scratch_types=[
            pltpu.VMEM((half * ECHUNK,), jnp.int32),
            pltpu.VMEM((half, ECHUNK), jnp.int32),
            pltpu.VMEM((nbuf, ECHUNK, d), jnp.float32),
            pltpu.VMEM_SHARED((npad, d), jnp.float32),
        ]
        + [pltpu.SemaphoreType.DMA] * nbuf,
    )
    def k(src_h, dst_h, hws_h, out_h, src_v, dst_v, rows_v, acc_sh, *sems):
        cid = lax.axis_index("c")
        sid = lax.axis_index("s")
        wid = sid * NC + cid

        @pl.loop(0, ECHUNK)
        def _(r):
            @pl.loop(0, d, step=16)
            def _(c):
                rows_v[0, r, pl.ds(c, 16)] = jnp.zeros((16,), jnp.float32)

        @pl.loop(0, rows_pt, step=ECHUNK)
        def _(r0):
            pltpu.sync_copy(
                rows_v.at[0], acc_sh.at[pl.ds(sid * rows_pt + r0, ECHUNK)]
            )

        plsc.subcore_barrier()

        def wait_gather(b):
            pltpu.make_async_copy(
                hws_h.at[pl.ds(0, ECHUNK)], rows_v.at[b], sems[b]
            ).wait()

        def src_idx(c):
            return src_v.at[pl.ds(c * ECHUNK, ECHUNK)]

        @pl.loop(0, npass)
        def _(p):
            pltpu.sync_copy(
                src_h.at[wid, pl.ds(p * half * ECHUNK, half * ECHUNK)], src_v
            )
            pltpu.sync_copy(dst_h.at[wid, pl.ds(p * half, half)], dst_v)
            for b in range(nbuf):
                pltpu.async_copy(hws_h.at[src_idx(b)], rows_v.at[b], sems[b])

            @pl.loop(0, half, step=nbuf)
            def _(j):
                for b in range(nbuf):
                    wait_gather(b)
                    pltpu.sync_copy(
                        rows_v.at[b], acc_sh.at[dst_v.at[j + b]], add=True
                    )

                    @pl.when(j + b + nbuf < half)
                    def _():
                        pltpu.async_copy(
                            hws_h.at[src_idx(j + b + nbuf)],
                            rows_v.at[b],
                            sems[b],
                        )

        plsc.subcore_barrier()
        sl = pl.ds(sid * rows_pt, rows_pt)
        pltpu.sync_copy(acc_sh.at[sl], out_h.at[cid, sl])

    return k(src3, dst3, hws)


def _mm_body(x_ref, w_ref, o_ref):
    o_ref[...] = jnp.dot(
        x_ref[...], w_ref[...], preferred_element_type=jnp.float32
    )


def _mm(x, w, *, br):
    npad, d_in = x.shape
    d_out = w.shape[1]
    return pl.pallas_call(
        _mm_body,
        grid=(npad // br,),
        in_specs=[
            pl.BlockSpec((br, d_in), lambda i: (i, 0)),
            pl.BlockSpec((d_in, d_out), lambda i: (0, 0)),
        ],
        out_specs=pl.BlockSpec((br, d_out), lambda i: (i, 0)),
        out_shape=jax.ShapeDtypeStruct((npad, d_out), jnp.float32),
    )(x, w)


def _mm1ns_body(ds_ref, dd_ref, x_ref, w_ref, hws_ref, ns_ref, nd_ref):
    s = ds_ref[...]
    d = dd_ref[...]
    ns = 1.0 / jnp.sqrt(jnp.maximum(s[0] + s[1], 1.0))
    nd = 1.0 / jnp.sqrt(jnp.maximum(d[0] + d[1], 1.0))
    ns_ref[...] = ns
    nd_ref[...] = nd
    hws_ref[...] = (
        jnp.dot(x_ref[...], w_ref[...], preferred_element_type=jnp.float32) * ns
    )


def _mm1ns(dps, dpd, x, w, *, br):
    npad, d_in = x.shape
    d_out = w.shape[1]
    return pl.pallas_call(
        _mm1ns_body,
        grid=(npad // br,),
        in_specs=[
            pl.BlockSpec((2, br, 1), lambda i: (0, i, 0)),
            pl.BlockSpec((2, br, 1), lambda i: (0, i, 0)),
            pl.BlockSpec((br, d_in), lambda i: (i, 0)),
            pl.BlockSpec((d_in, d_out), lambda i: (0, 0)),
        ],
        out_specs=[
            pl.BlockSpec((br, d_out), lambda i: (i, 0)),
            pl.BlockSpec((br, 1), lambda i: (i, 0)),
            pl.BlockSpec((br, 1), lambda i: (i, 0)),
        ],
        out_shape=[
            jax.ShapeDtypeStruct((npad, d_out), jnp.float32),
            jax.ShapeDtypeStruct((npad, 1), jnp.float32),
            jax.ShapeDtypeStruct((npad, 1), jnp.float32),
        ],
    )(dps, dpd, x, w)


def _layer_body(p_ref, nd_ref, b_ref, w_ref, ns_ref, o_ref):
    p = p_ref[...]
    agg = (p[0] + p[1]) * nd_ref[...] + b_ref[...]
    x = jnp.maximum(agg, 0.0)
    o_ref[...] = (
        jnp.dot(x, w_ref[...], preferred_element_type=jnp.float32) * ns_ref[...]
    )


def _layer(p, nd, b, w, ns, *, br):
    _, npad, d_in = p.shape
    d_out = w.shape[1]
    return pl.pallas_call(
        _layer_body,
        grid=(npad // br,),
        in_specs=[
            pl.BlockSpec((2, br, d_in), lambda i: (0, i, 0)),
            pl.BlockSpec((br, 1), lambda i: (i, 0)),
            pl.BlockSpec((1, d_in), lambda i: (0, 0)),
            pl.BlockSpec((d_in, d_out), lambda i: (0, 0)),
            pl.BlockSpec((br, 1), lambda i: (i, 0)),
        ],
        out_specs=pl.BlockSpec((br, d_out), lambda i: (i, 0)),
        out_shape=jax.ShapeDtypeStruct((npad, d_out), jnp.float32),
    )(p, nd, b, w, ns)


def _scale_relu_body(p_ref, nd_ref, b_ref, ns_ref, o_ref):
    p = p_ref[...]
    agg = (p[0] + p[1]) * nd_ref[...] + b_ref[...]
    o_ref[...] = jnp.maximum(agg, 0.0) * ns_ref[...]


def _scale_relu(p, nd, b, ns, *, br):
    _, npad, d = p.shape
    return pl.pallas_call(
        _scale_relu_body,
        grid=(npad // br,),
        in_specs=[
            pl.BlockSpec((2, br, d), lambda i: (0, i, 0)),
            pl.BlockSpec((br, 1), lambda i: (i, 0)),
            pl.BlockSpec((1, d), lambda i: (0, 0)),
            pl.BlockSpec((br, 1), lambda i: (i, 0)),
        ],
        out_specs=pl.BlockSpec((br, d), lambda i: (i, 0)),
        out_shape=jax.ShapeDtypeStruct((npad, d), jnp.float32),
    )(p, nd, b, ns)


def _final_mm_body(p_ref, nd_ref, w_ref, b_ref, o_ref):
    p = p_ref[...]
    agg = (p[0] + p[1]) * nd_ref[...]
    o_ref[...] = (
        jnp.dot(agg, w_ref[...], preferred_element_type=jnp.float32) + b_ref[...]
    )


def _final_mm(p, nd, w, b, *, br):
    _, npad, d_in = p.shape
    d_out = w.shape[1]
    return pl.pallas_call(
        _final_mm_body,
        grid=(npad // br,),
        in_specs=[
            pl.BlockSpec((2, br, d_in), lambda i: (0, i, 0)),
            pl.BlockSpec((br, 1), lambda i: (i, 0)),
            pl.BlockSpec((d_in, d_out), lambda i: (0, 0)),
            pl.BlockSpec((1, d_out), lambda i: (0, 0)),
        ],
        out_specs=pl.BlockSpec((br, d_out), lambda i: (i, 0)),
        out_shape=jax.ShapeDtypeStruct((npad, d_out), jnp.float32),
    )(p, nd, w, b)


def kernel(h, edge_index, W1, b1, W2, b2, W3, b3):
    n, d_in = h.shape
    e = edge_index.shape[1]
    d_h = W2.shape[0]
    d_out = W3.shape[1]

    npad = _cdiv(n + 1, NS * CHUNK) * NS * CHUNK
    tile_edges = _cdiv(e, NW * ECHUNK * 8) * ECHUNK * 8
    epad = NW * tile_edges
    chunks_d = tile_edges // CHUNK
    chunks_e = tile_edges // ECHUNK
    pad_rows = npad - n

    src = edge_index[0]
    dst = edge_index[1]
    pad_idx = (n + (jnp.arange(epad - e, dtype=jnp.int32) % pad_rows)).astype(
        jnp.int32
    )
    src_p = jnp.concatenate([src, pad_idx])
    dst_p = jnp.concatenate([dst, pad_idx])
    src3d = src_p.reshape(NW, chunks_d, CHUNK)
    dst3d = dst_p.reshape(NW, chunks_d, CHUNK)
    src2e = src_p.reshape(NW, tile_edges)
    dst3e = dst_p.reshape(NW, chunks_e, ECHUNK)
    hp = jnp.zeros((npad, d_in), jnp.float32).at[:n].set(h)

    br = 2048

    dp = _degrees(src3d, dst3d, npad=npad, chunks=chunks_d)
    dps = dp[:, 0].reshape(NC, npad, 1)
    dpd = dp[:, 1].reshape(NC, npad, 1)
    hws1, ns, nd = _mm1ns(dps, dpd, hp, W1, br=br)

    p1 = _edge_aggregate(src2e, dst3e, hws1, npad=npad, d=d_h, chunks=chunks_e)
    hws2 = _layer(p1, nd, b1.reshape(1, d_h), W2, ns, br=br)
    p2 = _edge_aggregate(src2e, dst3e, hws2, npad=npad, d=d_h, chunks=chunks_e)
    x3s = _scale_relu(p2, nd, b2.reshape(1, d_h), ns, br=br)
    p3 = _edge_aggregate(src2e, dst3e, x3s, npad=npad, d=d_h, chunks=chunks_e)
    out = _final_mm(p3, nd, W3, b3.reshape(1, d_out), br=br)
    return out[:n]


# speedup vs baseline: 1.0797x; 1.0797x over previous
import functools

import jax
import jax.numpy as jnp
from jax import lax
from jax.experimental import pallas as pl
from jax.experimental.pallas import tpu as pltpu
from jax.experimental.pallas import tpu_sc as plsc

NC = 2
NS = 16
NW = NC * NS
CHUNK = 128
ECHUNK = 64


def _cdiv(a, b):
    return (a + b - 1) // b


def _sc_mesh():
    return plsc.VectorSubcoreMesh(core_axis_name="c", subcore_axis_name="s")


def _degrees(src3, dst3, *, npad, chunks):
    rows_pt = npad // NS

    @functools.partial(
        pl.kernel,
        mesh=_sc_mesh(),
        out_type=jax.ShapeDtypeStruct((NC, 2, npad), jnp.float32),
        scratch_types=[
            pltpu.VMEM((chunks, CHUNK), jnp.int32),
            pltpu.VMEM((chunks, CHUNK), jnp.int32),
            pltpu.VMEM((CHUNK,), jnp.float32),
            pltpu.VMEM((rows_pt,), jnp.float32),
            pltpu.VMEM_SHARED((npad,), jnp.float32),
            pltpu.VMEM_SHARED((npad,), jnp.float32),
            pltpu.SemaphoreType.DMA,
            pltpu.SemaphoreType.DMA,
        ],
    )
    def k(src_h, dst_h, out_h, src_v, dst_v, ones_v, z_v, degs_sh, degd_sh,
          sem_s, sem_d):
        cid = lax.axis_index("c")
        sid = lax.axis_index("s")
        wid = sid * NC + cid

        @pl.loop(0, CHUNK, step=16)
        def _(i):
            ones_v[pl.ds(i, 16)] = jnp.full((16,), 1.0, jnp.float32)

        @pl.loop(0, rows_pt, step=16)
        def _(i):
            z_v[pl.ds(i, 16)] = jnp.zeros((16,), jnp.float32)

        pltpu.sync_copy(z_v, degs_sh.at[pl.ds(sid * rows_pt, rows_pt)])
        pltpu.sync_copy(z_v, degd_sh.at[pl.ds(sid * rows_pt, rows_pt)])
        plsc.subcore_barrier()

        pltpu.sync_copy(src_h.at[wid], src_v)
        pltpu.sync_copy(dst_h.at[wid], dst_v)

        @pl.loop(0, chunks)
        def _(j):
            pltpu.async_copy(ones_v, degs_sh.at[src_v.at[j]], sem_s, add=True)
            pltpu.async_copy(ones_v, degd_sh.at[dst_v.at[j]], sem_d, add=True)

        @pl.loop(0, chunks)
        def _(j):
            pltpu.make_async_copy(ones_v, degs_sh.at[pl.ds(0, CHUNK)], sem_s).wait()
            pltpu.make_async_copy(ones_v, degd_sh.at[pl.ds(0, CHUNK)], sem_d).wait()

        plsc.subcore_barrier()
        sl = pl.ds(sid * rows_pt, rows_pt)
        pltpu.sync_copy(degs_sh.at[sl], out_h.at[cid, 0, sl])
        pltpu.sync_copy(degd_sh.at[sl], out_h.at[cid, 1, sl])

    return k(src3, dst3)


def _edge_aggregate(src3, dst3, hws, *, npad, d, chunks):
    rows_pt = npad // NS

    nbuf = 4
    npass = 2
    assert chunks % (nbuf * npass) == 0
    half = chunks // npass

    @functools.partial(
        pl.kernel,
        mesh=_sc_mesh(),
        out_type=jax.ShapeDtypeStruct((NC, npad, d), jnp.float32),
        scratch_types=[
            pltpu.VMEM((half * ECHUNK,), jnp.int32),
            pltpu.VMEM((half, ECHUNK), jnp.int32),
            pltpu.VMEM((nbuf, ECHUNK, d), jnp.float32),
            pltpu.VMEM_SHARED((npad, d), jnp.float32),
        ]
        + [pltpu.SemaphoreType.DMA] * nbuf,
    )
    def k(src_h, dst_h, hws_h, out_h, src_v, dst_v, rows_v, acc_sh, *sems):
        cid = lax.axis_index("c")
        sid = lax.axis_index("s")
        wid = sid * NC + cid

        @pl.loop(0, ECHUNK)
        def _(r):
            @pl.loop(0, d, step=16)
            def _(c):
                rows_v[0, r, pl.ds(c, 16)] = jnp.zeros((16,), jnp.float32)

        @pl.loop(0, rows_pt, step=ECHUNK)
        def _(r0):
            pltpu.sync_copy(
                rows_v.at[0], acc_sh.at[pl.ds(sid * rows_pt + r0, ECHUNK)]
            )

        plsc.subcore_barrier()

        def wait_gather(b):
            pltpu.make_async_copy(
                hws_h.at[pl.ds(0, ECHUNK)], rows_v.at[b], sems[b]
            ).wait()

        def src_idx(c):
            return src_v.at[pl.ds(c * ECHUNK, ECHUNK)]

        @pl.loop(0, npass)
        def _(p):
            pltpu.sync_copy(
                src_h.at[wid, pl.ds(p * half * ECHUNK, half * ECHUNK)], src_v
            )
            pltpu.sync_copy(dst_h.at[wid, pl.ds(p * half, half)], dst_v)
            for b in range(nbuf):
                pltpu.async_copy(hws_h.at[src_idx(b)], rows_v.at[b], sems[b])

            @pl.loop(0, half, step=nbuf)
            def _(j):
                for b in range(nbuf):
                    wait_gather(b)
                    pltpu.sync_copy(
                        rows_v.at[b], acc_sh.at[dst_v.at[j + b]], add=True
                    )

                    @pl.when(j + b + nbuf < half)
                    def _():
                        pltpu.async_copy(
                            hws_h.at[src_idx(j + b + nbuf)],
                            rows_v.at[b],
                            sems[b],
                        )

        plsc.subcore_barrier()
        sl = pl.ds(sid * rows_pt, rows_pt)
        pltpu.sync_copy(acc_sh.at[sl], out_h.at[cid, sl])

    return k(src3, dst3, hws)


def _mm_body(x_ref, w_ref, o_ref):
    o_ref[...] = jnp.dot(
        x_ref[...], w_ref[...], preferred_element_type=jnp.float32
    )


def _mm(x, w, *, br):
    npad, d_in = x.shape
    d_out = w.shape[1]
    return pl.pallas_call(
        _mm_body,
        grid=(npad // br,),
        in_specs=[
            pl.BlockSpec((br, d_in), lambda i: (i, 0)),
            pl.BlockSpec((d_in, d_out), lambda i: (0, 0)),
        ],
        out_specs=pl.BlockSpec((br, d_out), lambda i: (i, 0)),
        out_shape=jax.ShapeDtypeStruct((npad, d_out), jnp.float32),
    )(x, w)


def _mm1ns_body(ds_ref, dd_ref, x_ref, w_ref, hws_ref, ns_ref, nd_ref):
    s = ds_ref[...]
    d = dd_ref[...]
    ns = 1.0 / jnp.sqrt(jnp.maximum(s[0] + s[1], 1.0))
    nd = 1.0 / jnp.sqrt(jnp.maximum(d[0] + d[1], 1.0))
    ns_ref[...] = ns
    nd_ref[...] = nd
    hws_ref[...] = (
        jnp.dot(x_ref[...], w_ref[...], preferred_element_type=jnp.float32) * ns
    )


def _mm1ns(dps, dpd, x, w, *, br):
    npad, d_in = x.shape
    d_out = w.shape[1]
    return pl.pallas_call(
        _mm1ns_body,
        grid=(npad // br,),
        in_specs=[
            pl.BlockSpec((2, br, 1), lambda i: (0, i, 0)),
            pl.BlockSpec((2, br, 1), lambda i: (0, i, 0)),
            pl.BlockSpec((br, d_in), lambda i: (i, 0)),
            pl.BlockSpec((d_in, d_out), lambda i: (0, 0)),
        ],
        out_specs=[
            pl.BlockSpec((br, d_out), lambda i: (i, 0)),
            pl.BlockSpec((br, 1), lambda i: (i, 0)),
            pl.BlockSpec((br, 1), lambda i: (i, 0)),
        ],
        out_shape=[
            jax.ShapeDtypeStruct((npad, d_out), jnp.float32),
            jax.ShapeDtypeStruct((npad, 1), jnp.float32),
            jax.ShapeDtypeStruct((npad, 1), jnp.float32),
        ],
    )(dps, dpd, x, w)


def _layer_body(p_ref, nd_ref, b_ref, w_ref, ns_ref, o_ref):
    p = p_ref[...]
    agg = (p[0] + p[1]) * nd_ref[...] + b_ref[...]
    x = jnp.maximum(agg, 0.0)
    o_ref[...] = (
        jnp.dot(x, w_ref[...], preferred_element_type=jnp.float32) * ns_ref[...]
    )


def _layer(p, nd, b, w, ns, *, br):
    _, npad, d_in = p.shape
    d_out = w.shape[1]
    return pl.pallas_call(
        _layer_body,
        grid=(npad // br,),
        in_specs=[
            pl.BlockSpec((2, br, d_in), lambda i: (0, i, 0)),
            pl.BlockSpec((br, 1), lambda i: (i, 0)),
            pl.BlockSpec((1, d_in), lambda i: (0, 0)),
            pl.BlockSpec((d_in, d_out), lambda i: (0, 0)),
            pl.BlockSpec((br, 1), lambda i: (i, 0)),
        ],
        out_specs=pl.BlockSpec((br, d_out), lambda i: (i, 0)),
        out_shape=jax.ShapeDtypeStruct((npad, d_out), jnp.float32),
    )(p, nd, b, w, ns)


def _scale_relu_body(p_ref, nd_ref, b_ref, ns_ref, o_ref):
    p = p_ref[...]
    agg = (p[0] + p[1]) * nd_ref[...] + b_ref[...]
    o_ref[...] = jnp.maximum(agg, 0.0) * ns_ref[...]


def _scale_relu(p, nd, b, ns, *, br):
    _, npad, d = p.shape
    return pl.pallas_call(
        _scale_relu_body,
        grid=(npad // br,),
        in_specs=[
            pl.BlockSpec((2, br, d), lambda i: (0, i, 0)),
            pl.BlockSpec((br, 1), lambda i: (i, 0)),
            pl.BlockSpec((1, d), lambda i: (0, 0)),
            pl.BlockSpec((br, 1), lambda i: (i, 0)),
        ],
        out_specs=pl.BlockSpec((br, d), lambda i: (i, 0)),
        out_shape=jax.ShapeDtypeStruct((npad, d), jnp.float32),
    )(p, nd, b, ns)


def _final_mm_body(p_ref, nd_ref, w_ref, b_ref, o_ref):
    p = p_ref[...]
    agg = (p[0] + p[1]) * nd_ref[...]
    o_ref[...] = (
        jnp.dot(agg, w_ref[...], preferred_element_type=jnp.float32) + b_ref[...]
    )


def _final_mm(p, nd, w, b, *, br):
    _, npad, d_in = p.shape
    d_out = w.shape[1]
    return pl.pallas_call(
        _final_mm_body,
        grid=(npad // br,),
        in_specs=[
            pl.BlockSpec((2, br, d_in), lambda i: (0, i, 0)),
            pl.BlockSpec((br, 1), lambda i: (i, 0)),
            pl.BlockSpec((d_in, d_out), lambda i: (0, 0)),
            pl.BlockSpec((1, d_out), lambda i: (0, 0)),
        ],
        out_specs=pl.BlockSpec((br, d_out), lambda i: (i, 0)),
        out_shape=jax.ShapeDtypeStruct((npad, d_out), jnp.float32),
    )(p, nd, w, b)


def kernel(h, edge_index, W1, b1, W2, b2, W3, b3):
    n, d_in = h.shape
    e = edge_index.shape[1]
    d_h = W2.shape[0]
    d_out = W3.shape[1]

    npad = _cdiv(n + 1, NS * CHUNK) * NS * CHUNK
    tile_edges = _cdiv(e, NW * ECHUNK * 8) * ECHUNK * 8
    epad = NW * tile_edges
    chunks_d = tile_edges // CHUNK
    chunks_e = tile_edges // ECHUNK
    pad_rows = npad - n

    src = edge_index[0]
    dst = edge_index[1]
    pad_idx = (n + (jnp.arange(epad - e, dtype=jnp.int32) % pad_rows)).astype(
        jnp.int32
    )
    src_p = jnp.concatenate([src, pad_idx])
    dst_p = jnp.concatenate([dst, pad_idx])
    src3d = src_p.reshape(NW, chunks_d, CHUNK)
    dst3d = dst_p.reshape(NW, chunks_d, CHUNK)
    src2e = src_p.reshape(NW, tile_edges)
    dst3e = dst_p.reshape(NW, chunks_e, ECHUNK)
    hp = jnp.zeros((npad, d_in), jnp.float32).at[:n].set(h)

    br = 2048

    dp = _degrees(src3d, dst3d, npad=npad, chunks=chunks_d)
    dps = dp[:, 0].reshape(NC, npad, 1)
    dpd = dp[:, 1].reshape(NC, npad, 1)
    hws1, ns, nd = _mm1ns(dps, dpd, hp, W1, br=br)

    p1 = _edge_aggregate(src2e, dst3e, hws1, npad=npad, d=d_h, chunks=chunks_e)
    hws2 = _layer(p1, nd, b1.reshape(1, d_h), W2, ns, br=br)
    p2 = _edge_aggregate(src2e, dst3e, hws2, npad=npad, d=d_h, chunks=chunks_e)
    x3s = _scale_relu(p2, nd, b2.reshape(1, d_h), ns, br=br)
    p3 = _edge_aggregate(src2e, dst3e, x3s, npad=npad, d=d_h, chunks=chunks_e)
    out = _final_mm(p3, nd, W3, b3.reshape(1, d_out), br=br)
    return out[:n]
